# speedup vs baseline: 10.7154x; 10.7154x over previous
import jax
import jax.numpy as jnp
from jax.experimental import pallas as pl
from jax.experimental.pallas import tpu as pltpu

N_TOK = 4096
N_CLU = 256
N_FEA = 128


def _body(x_ref, c_ref, out_ref):
    x = x_ref[:]
    c = c_ref[:]
    xn = jnp.sum(x * x, axis=1, keepdims=True)
    xc = jax.lax.dot_general(
        x, c, (((1,), (1,)), ((), ())),
        preferred_element_type=jnp.float32,
        precision=jax.lax.Precision.HIGHEST,
    )
    scores = xn - 2.0 * xc
    m = jnp.min(scores, axis=0, keepdims=True)
    rows = jax.lax.broadcasted_iota(jnp.int32, (N_TOK, N_CLU), 0)
    idx = jnp.min(jnp.where(scores == m, rows, N_TOK), axis=0)
    cols = jax.lax.broadcasted_iota(jnp.int32, (N_CLU, N_TOK), 1)
    onehot = (cols == idx[:, None]).astype(jnp.float32)
    out_ref[:] = jax.lax.dot_general(
        onehot, x, (((1,), (0,)), ((), ())),
        preferred_element_type=jnp.float32,
        precision=jax.lax.Precision.HIGHEST,
    )


def kernel(x, cluster_centers):
    x2 = x.reshape(N_TOK, N_FEA)
    out = pl.pallas_call(
        _body,
        out_shape=jax.ShapeDtypeStruct((N_CLU, N_FEA), jnp.float32),
    )(x2, cluster_centers)
    return out[None]
